# baseline (speedup 1.0000x reference)
import functools

import jax
import jax.numpy as jnp
from jax import lax
from jax.experimental import pallas as pl
from jax.experimental.pallas import tpu as pltpu
from jax.experimental.pallas import tpu_sc as plsc

VOCAB = 1000000
DIM = 64
BATCH = 4096
SEQ = 200

NUM_WORKERS = 32
ROWS = BATCH * SEQ
PER_W = ROWS // NUM_WORKERS
CHUNK = 800
NCHUNK = PER_W // CHUNK
SUB = 100
NSUB = CHUNK // SUB
REPS = CHUNK // SEQ


def _pos_encoding():
    pos = jnp.arange(SEQ, dtype=jnp.float32)
    denom = 10000.0 ** jnp.linspace(0.0, 1.0, DIM)
    arg = pos[:, None] / denom[None, :]
    col = jnp.arange(DIM)
    return jnp.where(col[None, :] % 2 == 0, jnp.sin(arg), jnp.cos(arg))


def _body(idx_hbm, table_hbm, pos_hbm, out_hbm, idx_v, rows_v, pos_v, sem):
    wid = lax.axis_index("s") * 2 + lax.axis_index("c")
    base_sub = wid * (PER_W // SUB)

    pltpu.sync_copy(pos_hbm, pos_v)

    def chunk_body(c, carry):
        r0 = base_sub + c * NSUB
        pltpu.sync_copy(idx_hbm.at[pl.ds(r0, NSUB)], idx_v)
        copies = []
        for j in range(NSUB):
            copies.append(
                pltpu.async_copy(
                    table_hbm.at[idx_v.at[j]],
                    rows_v.at[pl.ds(j * SUB, SUB)],
                    sem,
                )
            )
        for cp in copies:
            cp.wait()

        def add_row(r, carry2):
            p0 = pos_v[r, pl.ds(0, 16)]
            p1 = pos_v[r, pl.ds(16, 16)]
            p2 = pos_v[r, pl.ds(32, 16)]
            p3 = pos_v[r, pl.ds(48, 16)]
            for rep in range(REPS):
                row = rep * SEQ + r
                rows_v[row, pl.ds(0, 16)] = rows_v[row, pl.ds(0, 16)] + p0
                rows_v[row, pl.ds(16, 16)] = rows_v[row, pl.ds(16, 16)] + p1
                rows_v[row, pl.ds(32, 16)] = rows_v[row, pl.ds(32, 16)] + p2
                rows_v[row, pl.ds(48, 16)] = rows_v[row, pl.ds(48, 16)] + p3
            return carry2

        lax.fori_loop(0, SEQ, add_row, 0)

        pltpu.sync_copy(rows_v, out_hbm.at[pl.ds(r0 * SUB, CHUNK)])
        return carry

    lax.fori_loop(0, NCHUNK, chunk_body, 0)


@jax.jit
def _run(idx2d, table, pos):
    mesh = plsc.VectorSubcoreMesh(core_axis_name="c", subcore_axis_name="s")
    f = functools.partial(
        pl.kernel,
        mesh=mesh,
        out_type=jax.ShapeDtypeStruct((ROWS, DIM), jnp.float32),
        scratch_types=[
            pltpu.VMEM((NSUB, SUB), jnp.int32),
            pltpu.VMEM((CHUNK, DIM), jnp.float32),
            pltpu.VMEM((SEQ, DIM), jnp.float32),
            pltpu.SemaphoreType.DMA,
        ],
        compiler_params=pltpu.CompilerParams(use_tc_tiling_on_sc=False),
    )(_body)
    return f(idx2d, table, pos)


def kernel(input, table):
    idx2d = input.reshape(ROWS // SUB, SUB)
    pos = _pos_encoding()
    out = _run(idx2d, table, pos)
    return out.reshape(BATCH, SEQ, DIM)
